# baseline (speedup 1.0000x reference)
import functools

import jax
import jax.numpy as jnp
from jax import lax
from jax.experimental import pallas as pl
from jax.experimental.pallas import tpu as pltpu
from jax.experimental.pallas import tpu_sc as plsc

_SCALE = 2048 ** -0.5
_BLK = 1024

_S = 8192
_D = 2048
_NC = 2
_NS = 16
_NW = _NC * _NS
_ELEMS = _S * _D
_PER_W = _ELEMS // _NW
_CHUNK = 65536
_NCHUNK = _PER_W // _CHUNK


def _tc_scale_copy(emb_ref, o_ref):
    o_ref[...] = emb_ref[...] * _SCALE


def _tc_kernel(emb):
    s, d = emb.shape
    return pl.pallas_call(
        _tc_scale_copy,
        grid=(s // _BLK,),
        in_specs=[pl.BlockSpec((_BLK, d), lambda i: (i, 0))],
        out_specs=pl.BlockSpec((_BLK, d), lambda i: (i, 0)),
        out_shape=jax.ShapeDtypeStruct((s, d), emb.dtype),
    )(emb)


def _sc_scale_body(emb_hbm, out_hbm, buf, sem):
    wid = lax.axis_index("s") * _NC + lax.axis_index("c")
    base = wid * _PER_W

    def chunk_body(g, carry):
        off = base + g * _CHUNK
        pltpu.async_copy(emb_hbm.at[pl.ds(off, _CHUNK)], buf, sem).wait()

        def mul_body(i, c):
            sl = pl.ds(i * 16, 16)
            buf[sl] = buf[sl] * _SCALE
            return c

        lax.fori_loop(0, _CHUNK // 16, mul_body, 0)
        pltpu.async_copy(buf, out_hbm.at[pl.ds(off, _CHUNK)], sem).wait()
        return carry

    lax.fori_loop(0, _NCHUNK, chunk_body, 0)


@functools.lru_cache(maxsize=None)
def _sc_scale_kernel():
    return pl.kernel(
        _sc_scale_body,
        mesh=plsc.VectorSubcoreMesh(
            core_axis_name="c", subcore_axis_name="s"
        ),
        out_type=jax.ShapeDtypeStruct((_ELEMS,), jnp.float32),
        scratch_types=[
            pltpu.VMEM((_CHUNK,), jnp.float32),
            pltpu.SemaphoreType.DMA,
        ],
    )


def kernel(x, emb):
    out = _sc_scale_kernel()(emb.reshape(_ELEMS))
    return out.reshape(1, _S, _D)


# speedup vs baseline: 1.9014x; 1.9014x over previous
import functools

import jax
import jax.numpy as jnp
from jax import lax
from jax.experimental import pallas as pl
from jax.experimental.pallas import tpu as pltpu
from jax.experimental.pallas import tpu_sc as plsc

_SCALE = 2048 ** -0.5
_BLK = 1024

_S = 8192
_D = 2048
_NC = 2
_NS = 16
_NW = _NC * _NS
_ELEMS = _S * _D
_PER_W = _ELEMS // _NW
_CHUNK = 32768
_NCHUNK = _PER_W // _CHUNK
_UNROLL = 8


def _tc_scale_copy(emb_ref, o_ref):
    o_ref[...] = emb_ref[...] * _SCALE


def _tc_kernel(emb):
    s, d = emb.shape
    return pl.pallas_call(
        _tc_scale_copy,
        grid=(s // _BLK,),
        in_specs=[pl.BlockSpec((_BLK, d), lambda i: (i, 0))],
        out_specs=pl.BlockSpec((_BLK, d), lambda i: (i, 0)),
        out_shape=jax.ShapeDtypeStruct((s, d), emb.dtype),
    )(emb)


def _sc_scale_body(emb_hbm, out_hbm, buf0, buf1, gs0, gs1, ss0, ss1):
    wid = lax.axis_index("s") * _NC + lax.axis_index("c")
    base = wid * _PER_W
    bufs = (buf0, buf1)
    gsems = (gs0, gs1)
    ssems = (ss0, ss1)

    def src(g):
        return emb_hbm.at[pl.ds(base + g * _CHUNK, _CHUNK)]

    def dst(g):
        return out_hbm.at[pl.ds(base + g * _CHUNK, _CHUNK)]

    gathers = {0: pltpu.async_copy(src(0), bufs[0], gsems[0])}
    scatters = {}
    for g in range(_NCHUNK):
        b = g & 1
        gathers[g].wait()
        if g + 1 < _NCHUNK:
            ob = (g + 1) & 1
            if g >= 1:
                scatters[g - 1].wait()
            gathers[g + 1] = pltpu.async_copy(src(g + 1), bufs[ob], gsems[ob])
        buf = bufs[b]

        def mul_body(i, c, buf=buf):
            for u in range(_UNROLL):
                sl = pl.ds(i * (16 * _UNROLL) + u * 16, 16)
                buf[sl] = buf[sl] * _SCALE
            return c

        lax.fori_loop(0, _CHUNK // (16 * _UNROLL), mul_body, 0)
        scatters[g] = pltpu.async_copy(buf, dst(g), ssems[b])
    scatters[_NCHUNK - 2].wait()
    scatters[_NCHUNK - 1].wait()


@functools.lru_cache(maxsize=None)
def _sc_scale_kernel():
    return pl.kernel(
        _sc_scale_body,
        mesh=plsc.VectorSubcoreMesh(
            core_axis_name="c", subcore_axis_name="s"
        ),
        out_type=jax.ShapeDtypeStruct((_ELEMS,), jnp.float32),
        scratch_types=[
            pltpu.VMEM((_CHUNK,), jnp.float32),
            pltpu.VMEM((_CHUNK,), jnp.float32),
            pltpu.SemaphoreType.DMA,
            pltpu.SemaphoreType.DMA,
            pltpu.SemaphoreType.DMA,
            pltpu.SemaphoreType.DMA,
        ],
    )


def kernel(x, emb):
    out = _sc_scale_kernel()(emb.reshape(_ELEMS))
    return out.reshape(1, _S, _D)


# speedup vs baseline: 1.9096x; 1.0043x over previous
import functools

import jax
import jax.numpy as jnp
from jax import lax
from jax.experimental import pallas as pl
from jax.experimental.pallas import tpu as pltpu
from jax.experimental.pallas import tpu_sc as plsc

_SCALE = 2048 ** -0.5
_BLK = 1024

_S = 8192
_D = 2048
_NC = 2
_NS = 16
_NW = _NC * _NS
_ELEMS = _S * _D
_PER_W = _ELEMS // _NW
_CHUNK = 32768
_NCHUNK = _PER_W // _CHUNK
_UNROLL = 8


def _tc_scale_copy(emb_ref, o_ref):
    o_ref[...] = emb_ref[...] * _SCALE


def _tc_kernel(emb):
    s, d = emb.shape
    return pl.pallas_call(
        _tc_scale_copy,
        grid=(s // _BLK,),
        in_specs=[pl.BlockSpec((_BLK, d), lambda i: (i, 0))],
        out_specs=pl.BlockSpec((_BLK, d), lambda i: (i, 0)),
        out_shape=jax.ShapeDtypeStruct((s, d), emb.dtype),
    )(emb)


def _sc_scale_body(emb_hbm, out_hbm, buf0, buf1, gs0, gs1, ss0, ss1):
    wid = lax.axis_index("s") * _NC + lax.axis_index("c")
    base = wid * _PER_W
    bufs = (buf0, buf1)
    gsems = (gs0, gs1)
    ssems = (ss0, ss1)

    def src(g):
        return emb_hbm.at[pl.ds(base + g * _CHUNK, _CHUNK)]

    def dst(g):
        return out_hbm.at[pl.ds(base + g * _CHUNK, _CHUNK)]

    gathers = {0: pltpu.async_copy(src(0), bufs[0], gsems[0])}
    scatters = {}
    for g in range(_NCHUNK):
        b = g & 1
        gathers[g].wait()
        if g + 1 < _NCHUNK:
            ob = (g + 1) & 1
            if g >= 1:
                scatters[g - 1].wait()
            gathers[g + 1] = pltpu.async_copy(src(g + 1), bufs[ob], gsems[ob])
        buf = bufs[b]

        @plsc.parallel_loop(0, _CHUNK, step=16, unroll=_UNROLL)
        def _mul_body(i, buf=buf):
            sl = pl.ds(i, 16)
            buf[sl] = buf[sl] * _SCALE
        scatters[g] = pltpu.async_copy(buf, dst(g), ssems[b])
    scatters[_NCHUNK - 2].wait()
    scatters[_NCHUNK - 1].wait()


@functools.lru_cache(maxsize=None)
def _sc_scale_kernel():
    return pl.kernel(
        _sc_scale_body,
        mesh=plsc.VectorSubcoreMesh(
            core_axis_name="c", subcore_axis_name="s"
        ),
        out_type=jax.ShapeDtypeStruct((_ELEMS,), jnp.float32),
        scratch_types=[
            pltpu.VMEM((_CHUNK,), jnp.float32),
            pltpu.VMEM((_CHUNK,), jnp.float32),
            pltpu.SemaphoreType.DMA,
            pltpu.SemaphoreType.DMA,
            pltpu.SemaphoreType.DMA,
            pltpu.SemaphoreType.DMA,
        ],
    )


def kernel(x, emb):
    out = _sc_scale_kernel()(emb.reshape(_ELEMS))
    return out.reshape(1, _S, _D)


# speedup vs baseline: 5.0943x; 2.6677x over previous
import functools

import jax
import jax.numpy as jnp
from jax import lax
from jax.experimental import pallas as pl
from jax.experimental.pallas import tpu as pltpu
from jax.experimental.pallas import tpu_sc as plsc

_SCALE = 2048 ** -0.5
_BLK = 1024

_S = 8192
_D = 2048
_NC = 2
_NS = 16
_NW = _NC * _NS
_ROWS_W = _S // _NW
_CH = 8
_NCH = _ROWS_W // _CH
_NBUF = 4
_AHEAD = _NBUF - 2
_UNROLL = 8


def _tc_scale_copy(emb_ref, o_ref):
    o_ref[...] = emb_ref[...] * _SCALE


def _tc_kernel(emb):
    s, d = emb.shape
    return pl.pallas_call(
        _tc_scale_copy,
        grid=(s // _BLK,),
        in_specs=[pl.BlockSpec((_BLK, d), lambda i: (i, 0))],
        out_specs=pl.BlockSpec((_BLK, d), lambda i: (i, 0)),
        out_shape=jax.ShapeDtypeStruct((s, d), emb.dtype),
    )(emb)


def _sc_scale_body(emb_hbm, out_hbm, *rest):
    bufs = rest[:_NBUF]
    gsems = rest[_NBUF:2 * _NBUF]
    ssems = rest[2 * _NBUF:3 * _NBUF]
    wid = lax.axis_index("s") * _NC + lax.axis_index("c")
    row0 = wid * _ROWS_W

    def src(g):
        return emb_hbm.at[pl.ds(row0 + g * _CH, _CH), :]

    def dst(g):
        return out_hbm.at[pl.ds(row0 + g * _CH, _CH), :]

    gathers = {}
    scatters = {}
    waited = set()
    for g in range(min(_AHEAD, _NCH)):
        gathers[g] = pltpu.async_copy(src(g), bufs[g % _NBUF], gsems[g % _NBUF])
    for g in range(_NCH):
        b = g % _NBUF
        gathers[g].wait()
        buf = bufs[b]

        @plsc.parallel_loop(0, _CH * _D, step=16, unroll=_UNROLL)
        def _mul_body(i, buf=buf):
            r = lax.shift_right_logical(i, 11)
            c = lax.bitwise_and(i, _D - 1)
            sl = pl.ds(pl.multiple_of(c, 16), 16)
            buf[r, sl] = buf[r, sl] * _SCALE

        scatters[g] = pltpu.async_copy(buf, dst(g), ssems[b])
        nxt = g + _AHEAD
        if nxt < _NCH:
            prev = nxt - _NBUF
            if prev >= 0:
                scatters[prev].wait()
                waited.add(prev)
            gathers[nxt] = pltpu.async_copy(
                src(nxt), bufs[nxt % _NBUF], gsems[nxt % _NBUF]
            )
    for g in range(_NCH):
        if g not in waited:
            scatters[g].wait()


@functools.lru_cache(maxsize=None)
def _sc_scale_kernel():
    return pl.kernel(
        _sc_scale_body,
        mesh=plsc.VectorSubcoreMesh(
            core_axis_name="c", subcore_axis_name="s"
        ),
        out_type=jax.ShapeDtypeStruct((_S, _D), jnp.float32),
        scratch_types=(
            [pltpu.VMEM((_CH, _D), jnp.float32)] * _NBUF
            + [pltpu.SemaphoreType.DMA] * (2 * _NBUF)
        ),
    )


def kernel(x, emb):
    out = _sc_scale_kernel()(emb)
    return out[None]


# speedup vs baseline: 5.2714x; 1.0348x over previous
import functools

import jax
import jax.numpy as jnp
from jax import lax
from jax.experimental import pallas as pl
from jax.experimental.pallas import tpu as pltpu
from jax.experimental.pallas import tpu_sc as plsc

_SCALE = 2048 ** -0.5
_BLK = 1024

_S = 8192
_D = 2048
_NC = 2
_NS = 16
_NW = _NC * _NS
_ROWS_W = _S // _NW
_CH = 8
_NCH = _ROWS_W // _CH
_NBUF = 6
_AHEAD = _NBUF - 2
_UNROLL = 8


def _tc_scale_copy(emb_ref, o_ref):
    o_ref[...] = emb_ref[...] * _SCALE


def _tc_kernel(emb):
    s, d = emb.shape
    return pl.pallas_call(
        _tc_scale_copy,
        grid=(s // _BLK,),
        in_specs=[pl.BlockSpec((_BLK, d), lambda i: (i, 0))],
        out_specs=pl.BlockSpec((_BLK, d), lambda i: (i, 0)),
        out_shape=jax.ShapeDtypeStruct((s, d), emb.dtype),
    )(emb)


def _sc_scale_body(emb_hbm, out_hbm, *rest):
    bufs = rest[:_NBUF]
    gsems = rest[_NBUF:2 * _NBUF]
    ssems = rest[2 * _NBUF:3 * _NBUF]
    wid = lax.axis_index("s") * _NC + lax.axis_index("c")
    row0 = wid * _ROWS_W

    def src(g):
        return emb_hbm.at[pl.ds(row0 + g * _CH, _CH), :]

    def dst(g):
        return out_hbm.at[pl.ds(row0 + g * _CH, _CH), :]

    gathers = {}
    scatters = {}
    waited = set()
    for g in range(min(_AHEAD, _NCH)):
        gathers[g] = pltpu.async_copy(src(g), bufs[g % _NBUF], gsems[g % _NBUF])
    for g in range(_NCH):
        b = g % _NBUF
        gathers[g].wait()
        buf = bufs[b]

        @plsc.parallel_loop(0, _CH * _D, step=16, unroll=_UNROLL)
        def _mul_body(i, buf=buf):
            r = lax.shift_right_logical(i, 11)
            c = lax.bitwise_and(i, _D - 1)
            sl = pl.ds(pl.multiple_of(c, 16), 16)
            buf[r, sl] = buf[r, sl] * _SCALE

        scatters[g] = pltpu.async_copy(buf, dst(g), ssems[b])
        nxt = g + _AHEAD
        if nxt < _NCH:
            prev = nxt - _NBUF
            if prev >= 0:
                scatters[prev].wait()
                waited.add(prev)
            gathers[nxt] = pltpu.async_copy(
                src(nxt), bufs[nxt % _NBUF], gsems[nxt % _NBUF]
            )
    for g in range(_NCH):
        if g not in waited:
            scatters[g].wait()


@functools.lru_cache(maxsize=None)
def _sc_scale_kernel():
    return pl.kernel(
        _sc_scale_body,
        mesh=plsc.VectorSubcoreMesh(
            core_axis_name="c", subcore_axis_name="s"
        ),
        out_type=jax.ShapeDtypeStruct((_S, _D), jnp.float32),
        scratch_types=(
            [pltpu.VMEM((_CH, _D), jnp.float32)] * _NBUF
            + [pltpu.SemaphoreType.DMA] * (2 * _NBUF)
        ),
    )


def kernel(x, emb):
    out = _sc_scale_kernel()(emb)
    return out[None]
